# Initial kernel scaffold; baseline (speedup 1.0000x reference)
import jax
import jax.numpy as jnp
from jax.experimental import pallas as pl


def kernel(user, item, embed_user, embed_item):
    raise NotImplementedError("write your pallas kernel here")


# speedup vs baseline: 1.1057x; 1.1057x over previous
import functools

import jax
import jax.numpy as jnp
from jax import lax
from jax.experimental import pallas as pl
from jax.experimental.pallas import tpu as pltpu
from jax.experimental.pallas import tpu_sc as plsc

BATCH = 16384
EMBED_DIM = 128
NUM_CORES = 2
NUM_SUBCORES = 16
NUM_WORKERS = NUM_CORES * NUM_SUBCORES
B_PER_W = BATCH // NUM_WORKERS
CHUNK = 128
NUM_CHUNKS = B_PER_W // CHUNK
GROUPS_PER_CHUNK = CHUNK // 16


def _body(user_ref, item_ref, eu_ref, ei_ref, out_ref,
          idx_u, idx_i, ubuf, ibuf, outv, tbuf, sem_u, sem_i):
    wid = lax.axis_index("c") * NUM_SUBCORES + lax.axis_index("s")
    base = pl.multiple_of(wid * B_PER_W, B_PER_W)

    iota = lax.iota(jnp.int32, 16)

    for g in range(NUM_CHUNKS):
        pltpu.sync_copy(user_ref.at[pl.ds(base + g * CHUNK, CHUNK)], idx_u)
        pltpu.sync_copy(item_ref.at[pl.ds(base + g * CHUNK, CHUNK)], idx_i)
        cu = pltpu.async_copy(eu_ref.at[idx_u], ubuf, sem_u)
        ci = pltpu.async_copy(ei_ref.at[idx_i], ibuf, sem_i)
        cu.wait()
        ci.wait()

        def group(t, _, g=g):
            b0 = t * 16
            for j in range(16):
                row = b0 + j
                acc = ubuf[row, pl.ds(0, 16)] * ibuf[row, pl.ds(0, 16)]
                for k in range(1, 8):
                    acc = acc + (ubuf[row, pl.ds(16 * k, 16)]
                                 * ibuf[row, pl.ds(16 * k, 16)])
                tbuf[pl.ds(16 * j, 16)] = acc
            row16 = iota * 16
            tot = plsc.load_gather(tbuf, [row16])
            for col in range(1, 16):
                tot = tot + plsc.load_gather(tbuf, [row16 + col])
            outv[pl.ds(g * CHUNK + b0, 16)] = tot
            return 0

        lax.fori_loop(0, GROUPS_PER_CHUNK, group, 0)

    pltpu.sync_copy(outv, out_ref.at[pl.ds(base, B_PER_W)])


@jax.jit
def _mf(user, item, embed_user, embed_item):
    mesh = plsc.VectorSubcoreMesh(
        core_axis_name="c", subcore_axis_name="s",
        num_cores=NUM_CORES, num_subcores=NUM_SUBCORES)
    return pl.kernel(
        _body,
        out_type=jax.ShapeDtypeStruct((BATCH,), jnp.float32),
        mesh=mesh,
        compiler_params=pltpu.CompilerParams(needs_layout_passes=False),
        scratch_types=[
            pltpu.VMEM((CHUNK,), jnp.int32),
            pltpu.VMEM((CHUNK,), jnp.int32),
            pltpu.VMEM((CHUNK, EMBED_DIM), jnp.float32),
            pltpu.VMEM((CHUNK, EMBED_DIM), jnp.float32),
            pltpu.VMEM((B_PER_W,), jnp.float32),
            pltpu.VMEM((256,), jnp.float32),
            pltpu.SemaphoreType.DMA,
            pltpu.SemaphoreType.DMA,
        ],
    )(user, item, embed_user, embed_item)


def kernel(user, item, embed_user, embed_item):
    return _mf(user.astype(jnp.int32), item.astype(jnp.int32),
               embed_user, embed_item)


# speedup vs baseline: 1.3372x; 1.2095x over previous
import functools

import jax
import jax.numpy as jnp
from jax import lax
from jax.experimental import pallas as pl
from jax.experimental.pallas import tpu as pltpu
from jax.experimental.pallas import tpu_sc as plsc

BATCH = 16384
EMBED_DIM = 128
NUM_CORES = 2
NUM_SUBCORES = 16
NUM_WORKERS = NUM_CORES * NUM_SUBCORES
B_PER_W = BATCH // NUM_WORKERS
CHUNK = 128
NUM_CHUNKS = B_PER_W // CHUNK
GROUPS_PER_CHUNK = CHUNK // 16


def _body(user_ref, item_ref, eu_ref, ei_ref, out_ref,
          idx_u, idx_i, ubuf0, ibuf0, ubuf1, ibuf1, outv, tbuf,
          sem_u0, sem_i0, sem_u1, sem_i1):
    wid = lax.axis_index("c") * NUM_SUBCORES + lax.axis_index("s")
    base = pl.multiple_of(wid * B_PER_W, B_PER_W)

    pltpu.sync_copy(user_ref.at[pl.ds(base, B_PER_W)], idx_u)
    pltpu.sync_copy(item_ref.at[pl.ds(base, B_PER_W)], idx_i)

    iota = lax.iota(jnp.int32, 16)
    slots = ((ubuf0, ibuf0, sem_u0, sem_i0),
             (ubuf1, ibuf1, sem_u1, sem_i1))

    def start(g):
        ubuf, ibuf, sem_u, sem_i = slots[g % 2]
        cu = pltpu.async_copy(
            eu_ref.at[idx_u.at[pl.ds(g * CHUNK, CHUNK)]], ubuf, sem_u)
        ci = pltpu.async_copy(
            ei_ref.at[idx_i.at[pl.ds(g * CHUNK, CHUNK)]], ibuf, sem_i)
        return cu, ci

    pending = start(0)
    for g in range(NUM_CHUNKS):
        nxt = start(g + 1) if g + 1 < NUM_CHUNKS else None
        pending[0].wait()
        pending[1].wait()
        ubuf, ibuf, _, _ = slots[g % 2]

        def group(t, _, ubuf=ubuf, ibuf=ibuf, g=g):
            b0 = t * 16
            for j in range(16):
                row = b0 + j
                acc = ubuf[row, pl.ds(0, 16)] * ibuf[row, pl.ds(0, 16)]
                for k in range(1, 8):
                    acc = acc + (ubuf[row, pl.ds(16 * k, 16)]
                                 * ibuf[row, pl.ds(16 * k, 16)])
                tbuf[pl.ds(16 * j, 16)] = acc
            row16 = iota * 16
            tot = plsc.load_gather(tbuf, [row16])
            for col in range(1, 16):
                tot = tot + plsc.load_gather(tbuf, [row16 + col])
            outv[pl.ds(g * CHUNK + b0, 16)] = tot
            return 0

        lax.fori_loop(0, GROUPS_PER_CHUNK, group, 0)
        pending = nxt

    pltpu.sync_copy(outv, out_ref.at[pl.ds(base, B_PER_W)])


@jax.jit
def _mf(user, item, embed_user, embed_item):
    mesh = plsc.VectorSubcoreMesh(
        core_axis_name="c", subcore_axis_name="s",
        num_cores=NUM_CORES, num_subcores=NUM_SUBCORES)
    return pl.kernel(
        _body,
        out_type=jax.ShapeDtypeStruct((BATCH,), jnp.float32),
        mesh=mesh,
        compiler_params=pltpu.CompilerParams(
            needs_layout_passes=False,
            disable_bounds_checks=True,
            disable_semaphore_checks=True,
        ),
        scratch_types=[
            pltpu.VMEM((B_PER_W,), jnp.int32),
            pltpu.VMEM((B_PER_W,), jnp.int32),
            pltpu.VMEM((CHUNK, EMBED_DIM), jnp.float32),
            pltpu.VMEM((CHUNK, EMBED_DIM), jnp.float32),
            pltpu.VMEM((CHUNK, EMBED_DIM), jnp.float32),
            pltpu.VMEM((CHUNK, EMBED_DIM), jnp.float32),
            pltpu.VMEM((B_PER_W,), jnp.float32),
            pltpu.VMEM((256,), jnp.float32),
            pltpu.SemaphoreType.DMA,
            pltpu.SemaphoreType.DMA,
            pltpu.SemaphoreType.DMA,
            pltpu.SemaphoreType.DMA,
        ],
    )(user, item, embed_user, embed_item)


def kernel(user, item, embed_user, embed_item):
    return _mf(user.astype(jnp.int32), item.astype(jnp.int32),
               embed_user, embed_item)
